# baseline (speedup 1.0000x reference)
import functools

import jax
import jax.numpy as jnp
from jax import lax
from jax.experimental import pallas as pl
from jax.experimental.pallas import tpu as pltpu
from jax.experimental.pallas import tpu_sc as plsc

_LAT = 128
_NODES = 10000
_EDGES = 320000
_NC = 2
_NS = 16
_NW = _NC * _NS
_EPW = _EDGES // _NW
_C = 80
_NCHUNK = _EPW // _C
_NBUF = 4
_ROWBLK = 1000
_HI16 = -65536


def _pq_body(enc_ref, h_ref, w1t_ref, b1_ref, p_ref, q_ref):
    enc = enc_ref[...]
    hh = h_ref[...]
    wa = w1t_ref[0 * _LAT:1 * _LAT, :]
    wb = w1t_ref[1 * _LAT:2 * _LAT, :]
    wc = w1t_ref[2 * _LAT:3 * _LAT, :]
    wd = w1t_ref[3 * _LAT:4 * _LAT, :]
    f32 = jnp.float32
    p = (jnp.dot(enc, wa, preferred_element_type=f32)
         + jnp.dot(hh, wb, preferred_element_type=f32)).astype(jnp.bfloat16)
    q = (jnp.dot(enc, wc, preferred_element_type=f32)
         + jnp.dot(hh, wd, preferred_element_type=f32)
         + b1_ref[...]).astype(jnp.bfloat16)

    def pack(x):
        lo = lax.bitcast_convert_type(x[:, :_LAT // 2], jnp.uint16).astype(jnp.int32)
        hi = lax.bitcast_convert_type(x[:, _LAT // 2:], jnp.uint16).astype(jnp.int32)
        return (hi << 16) | lo

    p_ref[...] = pack(p)
    q_ref[...] = pack(q)


def _compute_pq(encoded, h, w1t, b1row):
    grid = (_NODES // _ROWBLK,)
    return pl.pallas_call(
        _pq_body,
        grid=grid,
        in_specs=[
            pl.BlockSpec((_ROWBLK, _LAT), lambda i: (i, 0)),
            pl.BlockSpec((_ROWBLK, _LAT), lambda i: (i, 0)),
            pl.BlockSpec((4 * _LAT, _LAT), lambda i: (0, 0)),
            pl.BlockSpec((1, _LAT), lambda i: (0, 0)),
        ],
        out_specs=[
            pl.BlockSpec((_ROWBLK, _LAT // 2), lambda i: (i, 0)),
            pl.BlockSpec((_ROWBLK, _LAT // 2), lambda i: (i, 0)),
        ],
        out_shape=[jax.ShapeDtypeStruct((_NODES, _LAT // 2), jnp.int32)] * 2,
    )(encoded, h, w1t, b1row)


def _lane_shuffle(v, idx):
    return lax.gather(
        v, idx[:, None],
        lax.GatherDimensionNumbers(
            offset_dims=(), collapsed_slice_dims=(0,), start_index_map=(0,)),
        slice_sizes=(1,), mode=lax.GatherScatterMode.PROMISE_IN_BOUNDS)


@functools.cache
def _make_edge_kernel():
    return functools.partial(
        pl.kernel,
        mesh=plsc.VectorSubcoreMesh(core_axis_name="c", subcore_axis_name="s"),
        compiler_params=pltpu.CompilerParams(use_tc_tiling_on_sc=False),
        out_type=jax.ShapeDtypeStruct((_EDGES,), jnp.float32),
        scratch_types=[
            pltpu.VMEM((_EPW,), jnp.int32),
            pltpu.VMEM((_EPW,), jnp.int32),
            pltpu.VMEM((_NBUF, _C, _LAT // 2), jnp.int32),
            pltpu.VMEM((_NBUF, _C, _LAT // 2), jnp.int32),
            pltpu.VMEM((_EPW,), jnp.float32),
            pltpu.VMEM((_LAT,), jnp.float32),
            pltpu.VMEM((16,), jnp.float32),
        ] + [pltpu.SemaphoreType.DMA] * (2 * _NBUF),
    )(_edge_body)


def _edge_body(p_hbm, q_hbm, src_hbm, dst_hbm, w2_hbm, b2_hbm, out_hbm,
               srcv, dstv, prows, qrows, outv, w2v, b2v, *allsems):
    wid = lax.axis_index("s") * _NC + lax.axis_index("c")
    base = wid * _EPW
    pltpu.sync_copy(src_hbm.at[pl.ds(base, _EPW)], srcv)
    pltpu.sync_copy(dst_hbm.at[pl.ds(base, _EPW)], dstv)
    pltpu.sync_copy(w2_hbm, w2v)
    pltpu.sync_copy(b2_hbm, b2v)
    w2k = []
    for k in range(4):
        w2k.append(w2v[pl.ds(k * 16, 16)])
        w2k.append(w2v[pl.ds(64 + k * 16, 16)])
    b2c = b2v[...]
    lane = lax.iota(jnp.int32, 16)
    bfly = [lane ^ (1 << s) for s in range(4)]
    sems = tuple((allsems[2 * b], allsems[2 * b + 1]) for b in range(_NBUF))

    def start(g, buf):
        off = g * _C
        sp, sq = sems[buf]
        pltpu.async_copy(p_hbm.at[srcv.at[pl.ds(off, _C)]], prows.at[buf], sp)
        pltpu.async_copy(q_hbm.at[dstv.at[pl.ds(off, _C)]], qrows.at[buf], sq)

    def finish(buf):
        sp, sq = sems[buf]
        pltpu.make_async_copy(p_hbm.at[srcv.at[pl.ds(0, _C)]], prows.at[buf], sp).wait()
        pltpu.make_async_copy(q_hbm.at[dstv.at[pl.ds(0, _C)]], qrows.at[buf], sq).wait()

    def compute(g, buf):
        off = g * _C
        prow = prows.at[buf]
        qrow = qrows.at[buf]

        @plsc.parallel_loop(0, _C // 16, unroll=2)
        def grp_body(t):
            eoff = t * 16
            res = jnp.zeros((16,), jnp.float32)
            for j in range(16):
                e = eoff + j
                acc = b2c
                for k in range(4):
                    vp = prow[e, pl.ds(k * 16, 16)]
                    vq = qrow[e, pl.ds(k * 16, 16)]
                    plo = lax.bitcast_convert_type(vp << 16, jnp.float32)
                    qlo = lax.bitcast_convert_type(vq << 16, jnp.float32)
                    phi = lax.bitcast_convert_type(vp & _HI16, jnp.float32)
                    qhi = lax.bitcast_convert_type(vq & _HI16, jnp.float32)
                    acc = acc + jnp.maximum(plo + qlo, 0.0) * w2k[2 * k]
                    acc = acc + jnp.maximum(phi + qhi, 0.0) * w2k[2 * k + 1]
                for s in range(4):
                    acc = acc + _lane_shuffle(acc, bfly[s])
                res = jnp.where(lane == j, acc, res)
            outv[pl.ds(off + eoff, 16)] = res

    for b in range(_NBUF - 1):
        start(b, b)

    def ring_body(i, carry):
        for b in range(_NBUF):
            g = _NBUF * i + b

            @pl.when(g + _NBUF - 1 < _NCHUNK)
            def _():
                start(g + _NBUF - 1, (b + _NBUF - 1) % _NBUF)

            finish(b)
            compute(g, b)
        return carry

    lax.fori_loop(0, _NCHUNK // _NBUF, ring_body, 0)
    for r in range(_NCHUNK % _NBUF):
        g = (_NCHUNK // _NBUF) * _NBUF + r
        finish(g % _NBUF)
        compute(g, g % _NBUF)
    pltpu.sync_copy(outv, out_hbm.at[pl.ds(base, _EPW)])


def kernel(encoded, h, edge_index, W1, b1, W2, b2):
    src = edge_index[0].astype(jnp.int32)
    dst = edge_index[1].astype(jnp.int32)
    w1t = W1.T
    b1row = b1.reshape(1, _LAT)
    pi, qi = _compute_pq(encoded, h, w1t, b1row)
    w2 = W2.reshape(_LAT)
    b2spread = jnp.full((16,), b2[0] / 16.0, dtype=jnp.float32)
    out = _make_edge_kernel()(pi, qi, src, dst, w2, b2spread)
    return out.reshape(_EDGES, 1)


# speedup vs baseline: 1.6506x; 1.6506x over previous
import functools

import jax
import jax.numpy as jnp
from jax import lax
from jax.experimental import pallas as pl
from jax.experimental.pallas import tpu as pltpu
from jax.experimental.pallas import tpu_sc as plsc

_LAT = 128
_NODES = 10000
_EDGES = 320000
_NC = 2
_NS = 16
_NW = _NC * _NS
_EPW = _EDGES // _NW
_C = 80
_NCHUNK = _EPW // _C
_NBUF = 4
_ROWBLK = 1000
_HI16 = -65536


def _pq_body(enc_ref, h_ref, w1t_ref, b1_ref, p_ref, q_ref):
    enc = enc_ref[...]
    hh = h_ref[...]
    wa = w1t_ref[0 * _LAT:1 * _LAT, :]
    wb = w1t_ref[1 * _LAT:2 * _LAT, :]
    wc = w1t_ref[2 * _LAT:3 * _LAT, :]
    wd = w1t_ref[3 * _LAT:4 * _LAT, :]
    f32 = jnp.float32
    p = (jnp.dot(enc, wa, preferred_element_type=f32)
         + jnp.dot(hh, wb, preferred_element_type=f32)).astype(jnp.bfloat16)
    q = (jnp.dot(enc, wc, preferred_element_type=f32)
         + jnp.dot(hh, wd, preferred_element_type=f32)
         + b1_ref[...]).astype(jnp.bfloat16)

    def pack(x):
        lo = lax.bitcast_convert_type(x[:, :_LAT // 2], jnp.uint16).astype(jnp.int32)
        hi = lax.bitcast_convert_type(x[:, _LAT // 2:], jnp.uint16).astype(jnp.int32)
        return (hi << 16) | lo

    p_ref[...] = pack(p)
    q_ref[...] = pack(q)


def _compute_pq(encoded, h, w1t, b1row):
    grid = (_NODES // _ROWBLK,)
    return pl.pallas_call(
        _pq_body,
        grid=grid,
        in_specs=[
            pl.BlockSpec((_ROWBLK, _LAT), lambda i: (i, 0)),
            pl.BlockSpec((_ROWBLK, _LAT), lambda i: (i, 0)),
            pl.BlockSpec((4 * _LAT, _LAT), lambda i: (0, 0)),
            pl.BlockSpec((1, _LAT), lambda i: (0, 0)),
        ],
        out_specs=[
            pl.BlockSpec((_ROWBLK, _LAT // 2), lambda i: (i, 0)),
            pl.BlockSpec((_ROWBLK, _LAT // 2), lambda i: (i, 0)),
        ],
        out_shape=[jax.ShapeDtypeStruct((_NODES, _LAT // 2), jnp.int32)] * 2,
    )(encoded, h, w1t, b1row)


def _lane_shuffle(v, idx):
    return lax.gather(
        v, idx[:, None],
        lax.GatherDimensionNumbers(
            offset_dims=(), collapsed_slice_dims=(0,), start_index_map=(0,)),
        slice_sizes=(1,), mode=lax.GatherScatterMode.PROMISE_IN_BOUNDS)


@functools.cache
def _make_edge_kernel():
    return functools.partial(
        pl.kernel,
        mesh=plsc.VectorSubcoreMesh(core_axis_name="c", subcore_axis_name="s"),
        compiler_params=pltpu.CompilerParams(use_tc_tiling_on_sc=False),
        out_type=jax.ShapeDtypeStruct((_EDGES,), jnp.float32),
        scratch_types=[
            pltpu.VMEM((_EPW,), jnp.int32),
            pltpu.VMEM((_EPW,), jnp.int32),
            pltpu.VMEM((_NBUF, _C, _LAT // 2), jnp.int32),
            pltpu.VMEM((_NBUF, _C, _LAT // 2), jnp.int32),
            pltpu.VMEM((_EPW,), jnp.float32),
            pltpu.VMEM((_LAT,), jnp.float32),
            pltpu.VMEM((16,), jnp.float32),
        ] + [pltpu.SemaphoreType.DMA] * (2 * _NBUF),
    )(_edge_body)


def _edge_body(p_hbm, q_hbm, src_hbm, dst_hbm, w2_hbm, b2_hbm, out_hbm,
               srcv, dstv, prows, qrows, outv, w2v, b2v, *allsems):
    wid = lax.axis_index("s") * _NC + lax.axis_index("c")
    base = wid * _EPW
    pltpu.sync_copy(src_hbm.at[pl.ds(base, _EPW)], srcv)
    pltpu.sync_copy(dst_hbm.at[pl.ds(base, _EPW)], dstv)
    pltpu.sync_copy(w2_hbm, w2v)
    pltpu.sync_copy(b2_hbm, b2v)
    w2k = []
    for k in range(4):
        w2k.append(w2v[pl.ds(k * 16, 16)])
        w2k.append(w2v[pl.ds(64 + k * 16, 16)])
    b2c = b2v[...]
    lane = lax.iota(jnp.int32, 16)
    bfly = [lane ^ (1 << s) for s in range(4)]
    sems = tuple((allsems[2 * b], allsems[2 * b + 1]) for b in range(_NBUF))

    def start(g, buf):
        off = g * _C
        sp, sq = sems[buf]
        pltpu.async_copy(p_hbm.at[srcv.at[pl.ds(off, _C)]], prows.at[buf], sp)
        pltpu.async_copy(q_hbm.at[dstv.at[pl.ds(off, _C)]], qrows.at[buf], sq)

    def finish(buf):
        sp, sq = sems[buf]
        pltpu.make_async_copy(p_hbm.at[srcv.at[pl.ds(0, _C)]], prows.at[buf], sp).wait()
        pltpu.make_async_copy(q_hbm.at[dstv.at[pl.ds(0, _C)]], qrows.at[buf], sq).wait()

    def compute(g, buf):
        off = g * _C
        prow = prows.at[buf]
        qrow = qrows.at[buf]

        def grp_body(t, gcarry):
            eoff = t * 16
            res = jnp.zeros((16,), jnp.float32)
            for j in range(16):
                e = eoff + j
                acc = b2c
                for k in range(4):
                    vp = prow[e, pl.ds(k * 16, 16)]
                    vq = qrow[e, pl.ds(k * 16, 16)]
                    plo = lax.bitcast_convert_type(vp << 16, jnp.float32)
                    qlo = lax.bitcast_convert_type(vq << 16, jnp.float32)
                    phi = lax.bitcast_convert_type(vp & _HI16, jnp.float32)
                    qhi = lax.bitcast_convert_type(vq & _HI16, jnp.float32)
                    acc = acc + jnp.maximum(plo + qlo, 0.0) * w2k[2 * k]
                    acc = acc + jnp.maximum(phi + qhi, 0.0) * w2k[2 * k + 1]
                for s in range(4):
                    acc = acc + _lane_shuffle(acc, bfly[s])
                res = jnp.where(lane == j, acc, res)
            outv[pl.ds(off + eoff, 16)] = res
            return gcarry

        lax.fori_loop(0, _C // 16, grp_body, 0)

    for b in range(_NBUF - 1):
        start(b, b)

    def ring_body(i, carry):
        for b in range(_NBUF):
            g = _NBUF * i + b

            @pl.when(g + _NBUF - 1 < _NCHUNK)
            def _():
                start(g + _NBUF - 1, (b + _NBUF - 1) % _NBUF)

            finish(b)
            compute(g, b)
        return carry

    lax.fori_loop(0, _NCHUNK // _NBUF, ring_body, 0)
    for r in range(_NCHUNK % _NBUF):
        g = (_NCHUNK // _NBUF) * _NBUF + r
        finish(g % _NBUF)
        compute(g, g % _NBUF)
    pltpu.sync_copy(outv, out_hbm.at[pl.ds(base, _EPW)])


def kernel(encoded, h, edge_index, W1, b1, W2, b2):
    src = edge_index[0].astype(jnp.int32)
    dst = edge_index[1].astype(jnp.int32)
    w1t = W1.T
    b1row = b1.reshape(1, _LAT)
    pi, qi = _compute_pq(encoded, h, w1t, b1row)
    w2 = W2.reshape(_LAT)
    b2spread = jnp.full((16,), b2[0] / 16.0, dtype=jnp.float32)
    out = _make_edge_kernel()(pi, qi, src, dst, w2, b2spread)
    return out.reshape(_EDGES, 1)
